# baseline (speedup 1.0000x reference)
import functools

import jax
import jax.numpy as jnp
from jax import lax
from jax.experimental import pallas as pl
from jax.experimental.pallas import tpu as pltpu
from jax.experimental.pallas import tpu_sc as plsc

N = 100000
D = 128
S = 512
U = 128
CHUNK = 256
NFULL = N // CHUNK
TAILA = 128
TAILB = N - NFULL * CHUNK - TAILA
NC = 2
NS = 16
NW = NC * NS
KBASE = NFULL // NW
NEXTRA = NFULL - KBASE * NW
NL = KBASE + 1
NBUF = 3
SPS = S // NS
L = 16


def _sc_body(x_hbm, batch_hbm, psum_hbm, pcnt_hbm,
             xbuf0, xbuf1, xbuf2, idx0, idx1, idx2, zbuf, cnt, ssum,
             semd0, semd1, semd2, sems0, sems1, sems2):
    cid = lax.axis_index("c")
    sid = lax.axis_index("s")
    w = cid * NS + sid

    xbufs = (xbuf0, xbuf1, xbuf2)
    idxs = (idx0, idx1, idx2)
    semds = (semd0, semd1, semd2)
    semss = (sems0, sems1, sems2)

    zeros16 = jnp.zeros((L,), jnp.float32)
    ones16 = jnp.ones((L,), jnp.float32)

    for i in range(S // L):
        cnt[pl.ds(i * L, L)] = zeros16

    def fill_z(i, _):
        for j in range(D // L):
            zbuf[i, pl.ds(j * L, L)] = zeros16
        return 0

    lax.fori_loop(0, SPS, fill_z, 0)

    pltpu.sync_copy(zbuf, ssum.at[pl.ds(sid * SPS, SPS), :])
    plsc.subcore_barrier()

    def start_dma(c, b):
        pltpu.async_copy(batch_hbm.at[pl.ds(c * CHUNK, U)],
                         idxs[b].at[0], semds[b])
        pltpu.async_copy(batch_hbm.at[pl.ds(c * CHUNK + U, U)],
                         idxs[b].at[1], semds[b])
        pltpu.async_copy(x_hbm.at[pl.ds(c * CHUNK, CHUNK), :], xbufs[b],
                         semds[b])

    def wait_dma(b):
        pltpu.make_async_copy(batch_hbm.at[pl.ds(0, U)], idxs[b].at[0],
                              semds[b]).wait()
        pltpu.make_async_copy(batch_hbm.at[pl.ds(0, U)], idxs[b].at[1],
                              semds[b]).wait()
        pltpu.make_async_copy(x_hbm.at[pl.ds(0, CHUNK), :], xbufs[b],
                              semds[b]).wait()

    def start_tail_dma(b):
        base = NFULL * CHUNK
        pltpu.async_copy(batch_hbm.at[pl.ds(base, TAILA)],
                         idxs[b].at[0], semds[b])
        pltpu.async_copy(batch_hbm.at[pl.ds(base + TAILA, TAILB)],
                         idxs[b].at[1, pl.ds(0, TAILB)], semds[b])
        pltpu.async_copy(x_hbm.at[pl.ds(base, TAILA + TAILB), :],
                         xbufs[b].at[pl.ds(0, TAILA + TAILB), :], semds[b])

    def wait_tail_dma(b):
        base = NFULL * CHUNK
        pltpu.make_async_copy(batch_hbm.at[pl.ds(base, TAILA)],
                              idxs[b].at[0], semds[b]).wait()
        pltpu.make_async_copy(batch_hbm.at[pl.ds(base, TAILB)],
                              idxs[b].at[1, pl.ds(0, TAILB)],
                              semds[b]).wait()
        pltpu.make_async_copy(x_hbm.at[pl.ds(base, TAILA + TAILB), :],
                              xbufs[b].at[pl.ds(0, TAILA + TAILB), :],
                              semds[b]).wait()

    def start_scatter(b):
        pltpu.async_copy(xbufs[b].at[pl.ds(0, U), :],
                         ssum.at[idxs[b].at[0]], semss[b], add=True)
        pltpu.async_copy(xbufs[b].at[pl.ds(U, U), :],
                         ssum.at[idxs[b].at[1]], semss[b], add=True)

    def wait_scatter(b):
        pltpu.make_async_copy(xbufs[b].at[pl.ds(0, U), :],
                              ssum.at[idxs[b].at[0]], semss[b]).wait()
        pltpu.make_async_copy(xbufs[b].at[pl.ds(U, U), :],
                              ssum.at[idxs[b].at[1]], semss[b]).wait()

    def histo(b, row, nv):
        for k in range(nv):
            idv = idxs[b][row, pl.ds(k * L, L)]
            plsc.addupdate_scatter(cnt, [idv], ones16)

    is_extra = w < NEXTRA
    is_tailw = w == NW - 1

    def start_logical(i, b):
        if i < KBASE:
            start_dma(w + NW * i, b)
        else:
            @pl.when(is_extra)
            def _():
                start_dma(w + NW * KBASE, b)

            @pl.when(is_tailw)
            def _():
                start_tail_dma(b)

    def process_logical(i, b):
        if i < KBASE:
            wait_dma(b)
            start_scatter(b)
            histo(b, 0, U // L)
            histo(b, 1, U // L)
        else:
            @pl.when(is_extra)
            def _():
                wait_dma(b)
                start_scatter(b)
                histo(b, 0, U // L)
                histo(b, 1, U // L)

            @pl.when(is_tailw)
            def _():
                wait_tail_dma(b)
                for k in range(TAILB // L, U // L):
                    idxs[b][1, pl.ds(k * L, L)] = jnp.full((L,), S - 1,
                                                           jnp.int32)

                def zero_row(r, _):
                    for j in range(D // L):
                        xbufs[b][r, pl.ds(j * L, L)] = zeros16
                    return 0

                lax.fori_loop(TAILA + TAILB, CHUNK, zero_row, 0)
                start_scatter(b)
                histo(b, 0, U // L)
                histo(b, 1, TAILB // L)

    def wait_scatter_logical(i, b):
        if i < KBASE:
            wait_scatter(b)
        else:
            @pl.when(jnp.logical_or(is_extra, is_tailw))
            def _():
                wait_scatter(b)

    start_logical(0, 0)
    start_logical(1, 1)
    for i in range(NL):
        b = i % NBUF
        if i + 2 < NL:
            if i + 2 >= NBUF:
                wait_scatter_logical(i + 2 - NBUF, (i + 2) % NBUF)
            start_logical(i + 2, (i + 2) % NBUF)
        process_logical(i, b)
    for i in range(max(0, NL - NBUF), NL):
        wait_scatter_logical(i, i % NBUF)

    pltpu.sync_copy(cnt, pcnt_hbm.at[w, :])

    plsc.subcore_barrier()

    pltpu.sync_copy(ssum.at[pl.ds(sid * SPS, SPS), :],
                    psum_hbm.at[cid, pl.ds(sid * SPS, SPS), :])


_sc_pool = functools.partial(
    pl.kernel,
    out_type=(jax.ShapeDtypeStruct((NC, S, D), jnp.float32),
              jax.ShapeDtypeStruct((NW, S), jnp.float32)),
    mesh=plsc.VectorSubcoreMesh(core_axis_name="c", subcore_axis_name="s"),
    scratch_types=[
        pltpu.VMEM((CHUNK, D), jnp.float32),
        pltpu.VMEM((CHUNK, D), jnp.float32),
        pltpu.VMEM((CHUNK, D), jnp.float32),
        pltpu.VMEM((2, U), jnp.int32),
        pltpu.VMEM((2, U), jnp.int32),
        pltpu.VMEM((2, U), jnp.int32),
        pltpu.VMEM((SPS, D), jnp.float32),
        pltpu.VMEM((S,), jnp.float32),
        pltpu.VMEM_SHARED((S, D), jnp.float32),
        pltpu.SemaphoreType.DMA,
        pltpu.SemaphoreType.DMA,
        pltpu.SemaphoreType.DMA,
        pltpu.SemaphoreType.DMA,
        pltpu.SemaphoreType.DMA,
        pltpu.SemaphoreType.DMA,
    ],
    compiler_params=pltpu.CompilerParams(needs_layout_passes=False,
                                         skip_device_barrier=True),
)(_sc_body)


def _combine_body(ps_ref, pc_ref, out_ref):
    sums = ps_ref[0] + ps_ref[1]
    cnt = jnp.sum(pc_ref[...], axis=0)
    cnt = jnp.maximum(cnt, 1.0)
    out_ref[...] = sums / cnt[:, None]


@jax.jit
def kernel(x, batch):
    psum, pcnt = _sc_pool(x, batch)
    return pl.pallas_call(
        _combine_body,
        out_shape=jax.ShapeDtypeStruct((S, D), jnp.float32),
    )(psum, pcnt)


# speedup vs baseline: 1.0000x; 1.0000x over previous
import functools

import jax
import jax.numpy as jnp
from jax import lax
from jax.experimental import pallas as pl
from jax.experimental.pallas import tpu as pltpu
from jax.experimental.pallas import tpu_sc as plsc

N = 100000
D = 128
S = 512
U = 128
CHUNK = 256
NFULL = N // CHUNK
TAILA = 128
TAILB = N - NFULL * CHUNK - TAILA
NC = 2
NS = 16
NW = NC * NS
KBASE = NFULL // NW
NEXTRA = NFULL - KBASE * NW
NL = KBASE + 1
NBUF = 3
SPS = S // NS
L = 16


def _sc_body(x_hbm, batch_hbm, psum_hbm, pcnt_hbm,
             xbuf0, xbuf1, xbuf2, idx0, idx1, idx2, zbuf, cnt, ssum,
             semd0, semd1, semd2, sems0, sems1, sems2):
    cid = lax.axis_index("c")
    sid = lax.axis_index("s")
    w = cid * NS + sid

    xbufs = (xbuf0, xbuf1, xbuf2)
    idxs = (idx0, idx1, idx2)
    semds = (semd0, semd1, semd2)
    semss = (sems0, sems1, sems2)

    zeros16 = jnp.zeros((L,), jnp.float32)
    ones16 = jnp.ones((L,), jnp.float32)

    for i in range(S // L):
        cnt[pl.ds(i * L, L)] = zeros16

    def fill_z(i, _):
        for j in range(D // L):
            zbuf[i, pl.ds(j * L, L)] = zeros16
        return 0

    lax.fori_loop(0, SPS, fill_z, 0)

    pltpu.sync_copy(zbuf, ssum.at[pl.ds(sid * SPS, SPS), :])
    plsc.subcore_barrier()

    def start_dma(c, b):
        pltpu.async_copy(batch_hbm.at[pl.ds(c * CHUNK, U)],
                         idxs[b].at[0], semds[b])
        pltpu.async_copy(batch_hbm.at[pl.ds(c * CHUNK + U, U)],
                         idxs[b].at[1], semds[b])
        pltpu.async_copy(x_hbm.at[pl.ds(c * CHUNK, CHUNK), :], xbufs[b],
                         semds[b])

    def wait_dma(b):
        pltpu.make_async_copy(batch_hbm.at[pl.ds(0, U)], idxs[b].at[0],
                              semds[b]).wait()
        pltpu.make_async_copy(batch_hbm.at[pl.ds(0, U)], idxs[b].at[1],
                              semds[b]).wait()
        pltpu.make_async_copy(x_hbm.at[pl.ds(0, CHUNK), :], xbufs[b],
                              semds[b]).wait()

    def start_tail_dma(b):
        base = NFULL * CHUNK
        pltpu.async_copy(batch_hbm.at[pl.ds(base, TAILA)],
                         idxs[b].at[0], semds[b])
        pltpu.async_copy(batch_hbm.at[pl.ds(base + TAILA, TAILB)],
                         idxs[b].at[1, pl.ds(0, TAILB)], semds[b])
        pltpu.async_copy(x_hbm.at[pl.ds(base, TAILA + TAILB), :],
                         xbufs[b].at[pl.ds(0, TAILA + TAILB), :], semds[b])

    def wait_tail_dma(b):
        base = NFULL * CHUNK
        pltpu.make_async_copy(batch_hbm.at[pl.ds(base, TAILA)],
                              idxs[b].at[0], semds[b]).wait()
        pltpu.make_async_copy(batch_hbm.at[pl.ds(base, TAILB)],
                              idxs[b].at[1, pl.ds(0, TAILB)],
                              semds[b]).wait()
        pltpu.make_async_copy(x_hbm.at[pl.ds(base, TAILA + TAILB), :],
                              xbufs[b].at[pl.ds(0, TAILA + TAILB), :],
                              semds[b]).wait()

    def start_scatter(b):
        pltpu.async_copy(xbufs[b].at[pl.ds(0, U), :],
                         ssum.at[idxs[b].at[0]], semss[b], add=True)
        pltpu.async_copy(xbufs[b].at[pl.ds(U, U), :],
                         ssum.at[idxs[b].at[1]], semss[b], add=True)

    def wait_scatter(b):
        pltpu.make_async_copy(xbufs[b].at[pl.ds(0, U), :],
                              ssum.at[idxs[b].at[0]], semss[b]).wait()
        pltpu.make_async_copy(xbufs[b].at[pl.ds(U, U), :],
                              ssum.at[idxs[b].at[1]], semss[b]).wait()

    def histo(b, row, nv):
        for k in range(nv):
            idv = idxs[b][row, pl.ds(k * L, L)]
            plsc.addupdate_scatter(cnt, [idv], ones16)

    is_extra = w < NEXTRA
    is_tailw = w == NW - 1

    def start_logical(i, b):
        if i < KBASE:
            start_dma(w + NW * i, b)
        else:
            @pl.when(is_extra)
            def _():
                start_dma(w + NW * KBASE, b)

            @pl.when(is_tailw)
            def _():
                start_tail_dma(b)

    def process_logical(i, b):
        if i < KBASE:
            wait_dma(b)
            start_scatter(b)
            histo(b, 0, U // L)
            histo(b, 1, U // L)
        else:
            @pl.when(is_extra)
            def _():
                wait_dma(b)
                start_scatter(b)
                histo(b, 0, U // L)
                histo(b, 1, U // L)

            @pl.when(is_tailw)
            def _():
                wait_tail_dma(b)
                for k in range(TAILB // L, U // L):
                    idxs[b][1, pl.ds(k * L, L)] = jnp.full((L,), S - 1,
                                                           jnp.int32)

                def zero_row(r, _):
                    for j in range(D // L):
                        xbufs[b][r, pl.ds(j * L, L)] = zeros16
                    return 0

                lax.fori_loop(TAILA + TAILB, CHUNK, zero_row, 0)
                start_scatter(b)
                histo(b, 0, U // L)
                histo(b, 1, TAILB // L)

    def wait_scatter_logical(i, b):
        if i < KBASE:
            wait_scatter(b)
        else:
            @pl.when(jnp.logical_or(is_extra, is_tailw))
            def _():
                wait_scatter(b)

    start_logical(0, 0)
    start_logical(1, 1)
    for i in range(NL):
        b = i % NBUF
        if i + 2 < NL:
            if i + 2 >= NBUF:
                wait_scatter_logical(i + 2 - NBUF, (i + 2) % NBUF)
            start_logical(i + 2, (i + 2) % NBUF)
        process_logical(i, b)
    for i in range(max(0, NL - NBUF), NL):
        wait_scatter_logical(i, i % NBUF)

    pltpu.sync_copy(cnt, pcnt_hbm.at[w, :])

    plsc.subcore_barrier()

    pltpu.sync_copy(ssum.at[pl.ds(sid * SPS, SPS), :],
                    psum_hbm.at[cid, pl.ds(sid * SPS, SPS), :])


_sc_pool = functools.partial(
    pl.kernel,
    out_type=(jax.ShapeDtypeStruct((NC, S, D), jnp.float32),
              jax.ShapeDtypeStruct((NW, S), jnp.float32)),
    mesh=plsc.VectorSubcoreMesh(core_axis_name="c", subcore_axis_name="s"),
    scratch_types=[
        pltpu.VMEM((CHUNK, D), jnp.float32),
        pltpu.VMEM((CHUNK, D), jnp.float32),
        pltpu.VMEM((CHUNK, D), jnp.float32),
        pltpu.VMEM((2, U), jnp.int32),
        pltpu.VMEM((2, U), jnp.int32),
        pltpu.VMEM((2, U), jnp.int32),
        pltpu.VMEM((SPS, D), jnp.float32),
        pltpu.VMEM((S,), jnp.float32),
        pltpu.VMEM_SHARED((S, D), jnp.float32),
        pltpu.SemaphoreType.DMA,
        pltpu.SemaphoreType.DMA,
        pltpu.SemaphoreType.DMA,
        pltpu.SemaphoreType.DMA,
        pltpu.SemaphoreType.DMA,
        pltpu.SemaphoreType.DMA,
    ],
    compiler_params=pltpu.CompilerParams(needs_layout_passes=False),
)(_sc_body)


def _combine_body(ps_ref, pc_ref, out_ref):
    sums = ps_ref[0] + ps_ref[1]
    cnt = jnp.sum(pc_ref[...], axis=0)
    cnt = jnp.maximum(cnt, 1.0)
    out_ref[...] = sums / cnt[:, None]


@jax.jit
def kernel(x, batch):
    psum, pcnt = _sc_pool(x, batch)
    return pl.pallas_call(
        _combine_body,
        out_shape=jax.ShapeDtypeStruct((S, D), jnp.float32),
    )(psum, pcnt)
